# baseline (device time: 112088 ns/iter reference)
import os

import jax
import jax.numpy as jnp
from jax import lax
from jax.experimental import pallas as pl
from jax.experimental.pallas import tpu as pltpu

_NO_COMM = bool(os.environ.get("KERNEL_NO_COMM"))
_NO_SOFTMAX = bool(os.environ.get("KERNEL_NO_SOFTMAX"))

N_DEV = 4
B_LOC = 2
SQ = 512
SKV = 512
HQ = 32
HPS = 8
DH = 64
D = 768
HD = HPS * DH

PERM = (0, 4, 1, 5, 2, 6, 3, 7)
NR = 4
RG = 2 * 64


def kernel(x, Wq, K_ext, V_ext, Wo):
    Wq16 = (Wq * 0.125).astype(jnp.bfloat16)
    Wo16 = Wo.astype(jnp.bfloat16)

    my_i = lax.axis_index("i")
    perm = list(PERM)
    x_p = x.reshape(B_LOC, 8, 64, D)[:, perm].reshape(B_LOC, SQ, D)
    K_loc = lax.dynamic_slice_in_dim(K_ext, my_i * B_LOC, B_LOC, axis=0)
    V_loc = lax.dynamic_slice_in_dim(V_ext, my_i * B_LOC, B_LOC, axis=0)

    def prep(kv):
        kv = kv.transpose(0, 2, 1, 3)
        kv = kv.reshape(B_LOC, HQ, 8, 64, DH)[:, :, perm]
        return kv.reshape(B_LOC * HQ, SKV, DH).astype(jnp.bfloat16)

    K2 = prep(K_loc)
    V2 = prep(V_loc)

    def body(x_ref, wq_ref, k_ref, v_ref, wo_ref, out_ref,
             wq_rbuf, wo_rbuf, ctx_ref, k_stage, v_stage,
             send_wq_sems, send_wo_sems, recv_wq_sems, recv_wo_sems,
             k_sems, v_sems):
        me = lax.axis_index("i")
        right = lax.rem(me + 1, N_DEV)
        left = lax.rem(me + 3, N_DEV)
        diag = lax.rem(me + 2, N_DEV)

        barrier_sem = pltpu.get_barrier_semaphore()
        for nbr in (right, left, diag):
            pl.semaphore_signal(
                barrier_sem, inc=1,
                device_id=(nbr,), device_id_type=pl.DeviceIdType.MESH,
            )
        pl.semaphore_wait(barrier_sem, 3)

        sends = []
        send_targets = () if _NO_COMM else ((right, 1), (left, 0), (diag, 2))
        for idx, (peer, slot) in enumerate(send_targets):
            for src, rbuf, ssems, rsems in (
                (wq_ref, wq_rbuf, send_wq_sems, recv_wq_sems),
                (wo_ref, wo_rbuf, send_wo_sems, recv_wo_sems),
            ):
                rdma = pltpu.make_async_remote_copy(
                    src_ref=src,
                    dst_ref=rbuf.at[slot],
                    send_sem=ssems.at[idx],
                    recv_sem=rsems.at[slot],
                    device_id=(peer,),
                    device_id_type=pl.DeviceIdType.MESH,
                )
                rdma.start()
                sends.append(rdma)

        x2 = x_ref[...].reshape(B_LOC * SQ, D).astype(jnp.bfloat16)

        def compute_group(j, wq, wo, first):
            cks, cvs = [], []
            for b in range(B_LOC):
                start = b * HQ + j * HPS
                ck = pltpu.make_async_copy(
                    k_ref.at[pl.ds(start, HPS)], k_stage.at[b], k_sems.at[b])
                cv = pltpu.make_async_copy(
                    v_ref.at[pl.ds(start, HPS)], v_stage.at[b], v_sems.at[b])
                ck.start()
                cv.start()
                cks.append(ck)
                cvs.append(cv)
            q2 = jnp.dot(x2, wq, preferred_element_type=jnp.float32)
            q16 = q2.astype(jnp.bfloat16)
            for b in range(B_LOC):
                cks[b].wait()
                cvs[b].wait()
                q16b = q16[b * SQ:(b + 1) * SQ]
                for h in range(HPS):
                    qh = q16b[:, h * DH:(h + 1) * DH]
                    kh = k_stage[b, h]
                    vh = v_stage[b, h]
                    for r in range(NR):
                        rs = r * RG
                        s = lax.dot_general(
                            qh[rs:rs + RG], kh[rs:rs + RG],
                            (((1,), (1,)), ((), ())),
                            preferred_element_type=jnp.float32,
                        )
                        if _NO_SOFTMAX:
                            e = s * 0.001
                        else:
                            e = jnp.exp(s)
                        denom = jnp.sum(e, axis=1, keepdims=True)
                        e16 = e.astype(jnp.bfloat16)
                        ctx = jnp.dot(e16, vh[rs:rs + RG],
                                      preferred_element_type=jnp.float32)
                        ctx = ctx * (1.0 / denom)
                        ctx_ref[b * SQ + rs:b * SQ + rs + RG,
                                h * DH:(h + 1) * DH] = ctx.astype(jnp.bfloat16)
            contrib = jnp.dot(
                ctx_ref[...], wo, preferred_element_type=jnp.float32
            )
            for b in range(B_LOC):
                for p in range(8):
                    src = contrib[b * SQ + 64 * p:b * SQ + 64 * (p + 1)]
                    dst = slice(64 * PERM[p], 64 * (PERM[p] + 1))
                    if first:
                        out_ref[b, dst, :] = src
                    else:
                        out_ref[b, dst, :] = out_ref[b, dst, :] + src

        compute_group(me, wq_ref[...], wo_ref[...], first=True)

        if _NO_COMM:
            for slot, j in ((0, right), (1, left), (2, diag)):
                compute_group(j, wq_ref[...], wo_ref[...], first=False)
            return

        for slot, j in ((0, right), (1, left), (2, diag)):
            for rbuf, ssems, rsems, dummy_src in (
                (wq_rbuf, send_wq_sems, recv_wq_sems, wq_ref),
                (wo_rbuf, send_wo_sems, recv_wo_sems, wo_ref),
            ):
                recv = pltpu.make_async_remote_copy(
                    src_ref=dummy_src,
                    dst_ref=rbuf.at[slot],
                    send_sem=ssems.at[0],
                    recv_sem=rsems.at[slot],
                    device_id=(me,),
                    device_id_type=pl.DeviceIdType.MESH,
                )
                recv.wait_recv()
            compute_group(j, wq_rbuf[slot], wo_rbuf[slot], first=False)

        for rdma in sends:
            rdma.wait_send()

    return pl.pallas_call(
        body,
        out_shape=jax.ShapeDtypeStruct((B_LOC, SQ, D), jnp.float32),
        in_specs=[
            pl.BlockSpec(memory_space=pltpu.MemorySpace.VMEM),
            pl.BlockSpec(memory_space=pltpu.MemorySpace.VMEM),
            pl.BlockSpec(memory_space=pl.ANY),
            pl.BlockSpec(memory_space=pl.ANY),
            pl.BlockSpec(memory_space=pltpu.MemorySpace.VMEM),
        ],
        out_specs=pl.BlockSpec(memory_space=pltpu.MemorySpace.VMEM),
        scratch_shapes=[
            pltpu.VMEM((3, D, HD), jnp.bfloat16),
            pltpu.VMEM((3, HD, D), jnp.bfloat16),
            pltpu.VMEM((B_LOC * SQ, HD), jnp.bfloat16),
            pltpu.VMEM((B_LOC, HPS, SKV, DH), jnp.bfloat16),
            pltpu.VMEM((B_LOC, HPS, SKV, DH), jnp.bfloat16),
            pltpu.SemaphoreType.DMA((3,)),
            pltpu.SemaphoreType.DMA((3,)),
            pltpu.SemaphoreType.DMA((3,)),
            pltpu.SemaphoreType.DMA((3,)),
            pltpu.SemaphoreType.DMA((2,)),
            pltpu.SemaphoreType.DMA((2,)),
        ],
        compiler_params=pltpu.CompilerParams(collective_id=0),
    )(x_p, Wq16, K2, V2, Wo16)


# device time: 63899 ns/iter; 1.7541x vs baseline; 1.7541x over previous
import os

import jax
import jax.numpy as jnp
from jax import lax
from jax.experimental import pallas as pl
from jax.experimental.pallas import tpu as pltpu

_NO_COMM = bool(os.environ.get("KERNEL_NO_COMM"))
_NO_SOFTMAX = bool(os.environ.get("KERNEL_NO_SOFTMAX"))

N_DEV = 4
B_LOC = 2
SQ = 512
SKV = 512
HQ = 32
HPS = 8
DH = 64
D = 768
HD = HPS * DH


def kernel(x, Wq, K_ext, V_ext, Wo):
    Wq16 = (Wq * 0.125).astype(jnp.bfloat16)
    Wo16 = Wo.astype(jnp.bfloat16)

    my_i = lax.axis_index("i")
    K_loc = lax.dynamic_slice_in_dim(K_ext, my_i * B_LOC, B_LOC, axis=0)
    V_loc = lax.dynamic_slice_in_dim(V_ext, my_i * B_LOC, B_LOC, axis=0)
    K2 = K_loc.transpose(0, 2, 1, 3).reshape(B_LOC * HQ, SKV, DH)
    V2 = V_loc.transpose(0, 2, 1, 3).reshape(B_LOC * HQ, SKV, DH)
    K2 = K2.astype(jnp.bfloat16)
    V2 = V2.astype(jnp.bfloat16)

    def body(x_ref, wq_ref, k_ref, v_ref, wo_ref, out_ref,
             wq_rbuf, wo_full, ctx_ref, k_stage, v_stage,
             send_wq_sems, send_wo_sems, recv_wq_sems, recv_wo_sems,
             k_sems, v_sems):
        me = lax.axis_index("i")
        right = lax.rem(me + 1, N_DEV)
        left = lax.rem(me + 3, N_DEV)
        diag = lax.rem(me + 2, N_DEV)

        barrier_sem = pltpu.get_barrier_semaphore()
        for nbr in (right, left, diag):
            pl.semaphore_signal(
                barrier_sem, inc=1,
                device_id=(nbr,), device_id_type=pl.DeviceIdType.MESH,
            )
        pl.semaphore_wait(barrier_sem, 3)

        sends = []
        send_targets = () if _NO_COMM else ((right, 1), (left, 0), (diag, 2))
        for idx, (peer, slot) in enumerate(send_targets):
            rdma_wq = pltpu.make_async_remote_copy(
                src_ref=wq_ref,
                dst_ref=wq_rbuf.at[slot],
                send_sem=send_wq_sems.at[idx],
                recv_sem=recv_wq_sems.at[slot],
                device_id=(peer,),
                device_id_type=pl.DeviceIdType.MESH,
            )
            rdma_wq.start()
            sends.append(rdma_wq)
            rdma_wo = pltpu.make_async_remote_copy(
                src_ref=wo_ref,
                dst_ref=wo_full.at[pl.ds((slot + 1) * HD, HD)],
                send_sem=send_wo_sems.at[idx],
                recv_sem=recv_wo_sems.at[slot],
                device_id=(peer,),
                device_id_type=pl.DeviceIdType.MESH,
            )
            rdma_wo.start()
            sends.append(rdma_wo)

        wo_full[pl.ds(0, HD)] = wo_ref[...]

        qi = lax.broadcasted_iota(jnp.int32, (SQ, SKV), 0)
        ki = lax.broadcasted_iota(jnp.int32, (SQ, SKV), 1)
        keep = (qi // 64) % 4 == (ki // 64) % 4
        maskf = keep.astype(jnp.float32)

        x2 = x_ref[...].reshape(B_LOC * SQ, D).astype(jnp.bfloat16)

        def compute_group(j, wq, step):
            cks, cvs = [], []
            for b in range(B_LOC):
                start = b * HQ + j * HPS
                ck = pltpu.make_async_copy(
                    k_ref.at[pl.ds(start, HPS)], k_stage.at[b], k_sems.at[b])
                cv = pltpu.make_async_copy(
                    v_ref.at[pl.ds(start, HPS)], v_stage.at[b], v_sems.at[b])
                ck.start()
                cv.start()
                cks.append(ck)
                cvs.append(cv)
            q2 = jnp.dot(x2, wq, preferred_element_type=jnp.float32)
            q16 = q2.astype(jnp.bfloat16)
            for b in range(B_LOC):
                cks[b].wait()
                cvs[b].wait()
                q16b = q16[b * SQ:(b + 1) * SQ]
                for h in range(HPS):
                    qh = q16b[:, h * DH:(h + 1) * DH]
                    s = lax.dot_general(
                        qh, k_stage[b, h], (((1,), (1,)), ((), ())),
                        preferred_element_type=jnp.float32,
                    )
                    if _NO_SOFTMAX:
                        e = s * 0.001
                    else:
                        e = jnp.exp(s) * maskf
                    denom = jnp.sum(e, axis=1, keepdims=True)
                    e16 = e.astype(jnp.bfloat16)
                    ctx = jnp.dot(e16, v_stage[b, h],
                                  preferred_element_type=jnp.float32)
                    ctx = ctx * (1.0 / denom)
                    ctx_ref[b * SQ:(b + 1) * SQ,
                            step * HD + h * DH:step * HD + (h + 1) * DH] = (
                        ctx.astype(jnp.bfloat16)
                    )

        compute_group(me, wq_ref[...], 0)

        if _NO_COMM:
            for step, j in ((1, right), (2, left), (3, diag)):
                compute_group(j, wq_ref[...], step)
        else:
            for slot, j in ((0, right), (1, left), (2, diag)):
                recv_wq = pltpu.make_async_remote_copy(
                    src_ref=wq_ref,
                    dst_ref=wq_rbuf.at[slot],
                    send_sem=send_wq_sems.at[0],
                    recv_sem=recv_wq_sems.at[slot],
                    device_id=(me,),
                    device_id_type=pl.DeviceIdType.MESH,
                )
                recv_wq.wait_recv()
                compute_group(j, wq_rbuf[slot], slot + 1)

            for slot in range(3):
                recv_wo = pltpu.make_async_remote_copy(
                    src_ref=wo_ref,
                    dst_ref=wo_full.at[pl.ds((slot + 1) * HD, HD)],
                    send_sem=send_wo_sems.at[0],
                    recv_sem=recv_wo_sems.at[slot],
                    device_id=(me,),
                    device_id_type=pl.DeviceIdType.MESH,
                )
                recv_wo.wait_recv()

        contrib = jnp.dot(
            ctx_ref[...], wo_full[...], preferred_element_type=jnp.float32
        )
        for b in range(B_LOC):
            out_ref[b, :, :] = contrib[b * SQ:(b + 1) * SQ]

        for rdma in sends:
            rdma.wait_send()

    return pl.pallas_call(
        body,
        out_shape=jax.ShapeDtypeStruct((B_LOC, SQ, D), jnp.float32),
        in_specs=[
            pl.BlockSpec(memory_space=pltpu.MemorySpace.VMEM),
            pl.BlockSpec(memory_space=pltpu.MemorySpace.VMEM),
            pl.BlockSpec(memory_space=pl.ANY),
            pl.BlockSpec(memory_space=pl.ANY),
            pl.BlockSpec(memory_space=pltpu.MemorySpace.VMEM),
        ],
        out_specs=pl.BlockSpec(memory_space=pltpu.MemorySpace.VMEM),
        scratch_shapes=[
            pltpu.VMEM((3, D, HD), jnp.bfloat16),
            pltpu.VMEM((N_DEV * HD, D), jnp.bfloat16),
            pltpu.VMEM((B_LOC * SQ, N_DEV * HD), jnp.bfloat16),
            pltpu.VMEM((B_LOC, HPS, SKV, DH), jnp.bfloat16),
            pltpu.VMEM((B_LOC, HPS, SKV, DH), jnp.bfloat16),
            pltpu.SemaphoreType.DMA((3,)),
            pltpu.SemaphoreType.DMA((3,)),
            pltpu.SemaphoreType.DMA((3,)),
            pltpu.SemaphoreType.DMA((3,)),
            pltpu.SemaphoreType.DMA((2,)),
            pltpu.SemaphoreType.DMA((2,)),
        ],
        compiler_params=pltpu.CompilerParams(collective_id=0),
    )(x, Wq16, K2, V2, Wo16)
